# baseline (speedup 1.0000x reference)
import functools

import jax
import jax.numpy as jnp
import numpy as np
from jax.experimental import pallas as pl
from jax.experimental.pallas import tpu as pltpu


def _build_input(ll_ref, lh_ref, hl_ref, hh_ref, fm_ref, t_ref,
                 eo_ref, fs_ref, xf_ref, *, C, Cf, Hh, Wh, CHK):
    W = 2 * Wh
    H = 2 * Hh
    Cin = C + Cf
    SCH = 2 * Hh + 2

    xf_ref[:, 0:W] = jnp.zeros((Cin, W), jnp.float32)
    xf_ref[:, (H + 1) * W:] = jnp.zeros((Cin, W), jnp.float32)

    sb_refs = (ll_ref, lh_ref, hl_ref, hh_ref)
    for ch in range(C // CHK):
        r0 = ch * CHK * Hh
        acc = None
        for s in range(4):
            x = sb_refs[s][0, r0:r0 + CHK * Hh, :]
            t = jnp.dot(x, t_ref[s * Wh:(s + 1) * Wh, :],
                        preferred_element_type=jnp.float32)
            acc = t if acc is None else acc + t
        for cl in range(CHK):
            rows = slice(cl * Hh, (cl + 1) * Hh)
            eo_ref[cl * SCH:cl * SCH + Hh, :] = acc[rows, 0:W]
            eo_ref[cl * SCH + Hh:cl * SCH + 2 * Hh, :] = acc[rows, W:2 * W]
        c0 = ch * CHK
        for i in range(Hh):
            off = (2 * i + 1) * W
            xf_ref[c0:c0 + CHK, off:off + W] = \
                eo_ref[i:i + CHK * SCH:SCH, :]
            xf_ref[c0:c0 + CHK, off + W:off + 2 * W] = \
                eo_ref[Hh + i:Hh + i + CHK * SCH:SCH, :]

    FCH = 16
    FS = FCH + 1
    for hc in range(H // FCH):
        h0 = hc * FCH
        for c in range(Cf):
            fs_ref[c * FS:c * FS + FCH, :] = \
                fm_ref[0, c * H + h0:c * H + h0 + FCH, :]
        for hl in range(FCH):
            y = h0 + hl
            xf_ref[C:Cin, (y + 1) * W:(y + 2) * W] = \
                fs_ref[hl:hl + Cf * FS:FS, :]


def _fused_kernel(ll_ref, lh_ref, hl_ref, hh_ref, fm_ref, w_ref, b_ref,
                  t_ref, m_ref, o_ref, eo_ref, fs_ref, xf_ref, *,
                  C, Cf, Hh, Wh, R, CHK):
    W = 2 * Wh
    H = 2 * Hh
    Cin = C + Cf
    SCH = 2 * Hh + 2

    @pl.when(pl.program_id(1) == 0)
    def _build():
        _build_input(ll_ref, lh_ref, hl_ref, hh_ref, fm_ref, t_ref,
                     eo_ref, fs_ref, xf_ref, C=C, Cf=Cf, Hh=Hh, Wh=Wh,
                     CHK=CHK)

    bias = b_ref[...]
    mask_l = m_ref[0:1, :]
    mask_r = m_ref[1:2, :]

    hb = pl.program_id(1)
    acc = None
    for kh in range(3):
        row_off = pl.multiple_of((hb * R + kh) * W, W)
        opnd = xf_ref[:, pl.ds(row_off, R * W)]
        for kw in range(3):
            t = jnp.dot(w_ref[3 * kh + kw], opnd,
                        preferred_element_type=jnp.float32)
            if kw == 0:
                t = jnp.roll(t, 1, axis=1) * mask_l
            elif kw == 2:
                t = jnp.roll(t, -1, axis=1) * mask_r
            acc = t if acc is None else acc + t
    o_ref[0, :, :] = jnp.maximum(acc + bias, 0.0)


def kernel(LL, LH, HL, HH, fm, conv_w, conv_b):
    N, C, Hh, Wh = LL.shape
    H, W = 2 * Hh, 2 * Wh
    Nf, Cf, Hf, Wf = fm.shape
    assert (Nf, Hf, Wf) == (N, H, W)
    Cout, Cin, kh, kw = conv_w.shape
    assert (kh, kw) == (3, 3) and Cin == C + Cf

    ll2 = LL.reshape(N, C * Hh, Wh).astype(jnp.float32)
    lh2 = LH.reshape(N, C * Hh, Wh).astype(jnp.float32)
    hl2 = HL.reshape(N, C * Hh, Wh).astype(jnp.float32)
    hh2 = HH.reshape(N, C * Hh, Wh).astype(jnp.float32)
    fm2 = fm.reshape(N, Cf * H, W).astype(jnp.float32)

    w9 = (jnp.asarray(conv_w, jnp.float32)
          .transpose(2, 3, 0, 1).reshape(9, Cout, Cin))
    b2 = jnp.asarray(conv_b, jnp.float32).reshape(Cout, 1)

    coef = np.array([[.5, -.5, -.5, .5],
                     [.5, -.5, .5, -.5],
                     [.5, .5, -.5, -.5],
                     [.5, .5, .5, .5]], np.float32)
    T = np.zeros((4 * Wh, 2 * W), np.float32)
    j = np.arange(Wh)
    for s in range(4):
        T[s * Wh + j, 2 * j] = coef[0, s]
        T[s * Wh + j, 2 * j + 1] = coef[1, s]
        T[s * Wh + j, W + 2 * j] = coef[2, s]
        T[s * Wh + j, W + 2 * j + 1] = coef[3, s]
    T = jnp.asarray(T)

    R = 16 if H % 16 == 0 else (8 if H % 8 == 0 else H)
    CHK = max(1, min(C, 256 // Hh))
    while C % CHK:
        CHK -= 1

    lane = np.arange(R * W) % W
    masks = np.stack([(lane != 0).astype(np.float32),
                      (lane != W - 1).astype(np.float32)])
    masks = jnp.asarray(masks)

    kernel_fn = functools.partial(_fused_kernel, C=C, Cf=Cf, Hh=Hh, Wh=Wh,
                                  R=R, CHK=CHK)
    sub_spec = pl.BlockSpec((1, C * Hh, Wh), lambda n, h: (n, 0, 0))
    out = pl.pallas_call(
        kernel_fn,
        out_shape=jax.ShapeDtypeStruct((N, Cout, H * W), jnp.float32),
        grid=(N, H // R),
        in_specs=[sub_spec, sub_spec, sub_spec, sub_spec,
                  pl.BlockSpec((1, Cf * H, W), lambda n, h: (n, 0, 0)),
                  pl.BlockSpec((9, Cout, Cin), lambda n, h: (0, 0, 0)),
                  pl.BlockSpec((Cout, 1), lambda n, h: (0, 0)),
                  pl.BlockSpec((4 * Wh, 2 * W), lambda n, h: (0, 0)),
                  pl.BlockSpec((2, R * W), lambda n, h: (0, 0))],
        out_specs=pl.BlockSpec((1, Cout, R * W), lambda n, h: (n, 0, h)),
        scratch_shapes=[pltpu.VMEM((CHK * (2 * Hh + 2), W), jnp.float32),
                        pltpu.VMEM((Cf * 17, W), jnp.float32),
                        pltpu.VMEM((C + Cf, (H + 2) * W), jnp.float32)],
        compiler_params=pltpu.CompilerParams(
            dimension_semantics=("parallel", "arbitrary")),
    )(ll2, lh2, hl2, hh2, fm2, w9, b2, T, masks)
    return out.reshape(N, Cout, H, W)


# speedup vs baseline: 1.2059x; 1.2059x over previous
import functools

import jax
import jax.numpy as jnp
import numpy as np
from jax.experimental import pallas as pl
from jax.experimental.pallas import tpu as pltpu


def _build_input(ll_ref, lh_ref, hl_ref, hh_ref, fm_ref, t_ref,
                 eo_ref, fs_ref, xf_ref, *, C, Cf, Hh, Wh, CHK):
    W = 2 * Wh
    H = 2 * Hh
    Cin = C + Cf
    SCH = 2 * Hh + 2

    def put(c0, nrows, off, val, width):
        xf_ref[c0:c0 + nrows, off:off + width] = val
        if off >= 2 * W:
            xf_ref[Cin + c0:Cin + c0 + nrows, off - 2 * W:off - 2 * W + width] = val

    zrow = jnp.zeros((Cin, W), jnp.float32)
    xf_ref[0:Cin, 0:W] = zrow
    xf_ref[Cin:2 * Cin, 0:W] = zrow
    put(0, Cin, (H + 1) * W, zrow, W)

    sb_refs = (ll_ref, lh_ref, hl_ref, hh_ref)
    for ch in range(C // CHK):
        r0 = ch * CHK * Hh
        acc = None
        for s in range(4):
            x = sb_refs[s][0, r0:r0 + CHK * Hh, :]
            t = jnp.dot(x, t_ref[s * Wh:(s + 1) * Wh, :],
                        preferred_element_type=jnp.float32)
            acc = t if acc is None else acc + t
        for cl in range(CHK):
            rows = slice(cl * Hh, (cl + 1) * Hh)
            eo_ref[cl * SCH:cl * SCH + Hh, :] = acc[rows, 0:W]
            eo_ref[cl * SCH + Hh:cl * SCH + 2 * Hh, :] = acc[rows, W:2 * W]
        c0 = ch * CHK
        for i in range(Hh):
            off = (2 * i + 1) * W
            put(c0, CHK, off, eo_ref[i:i + CHK * SCH:SCH, :], W)
            put(c0, CHK, off + W,
                eo_ref[Hh + i:Hh + i + CHK * SCH:SCH, :], W)

    FCH = 16
    FS = FCH + 1
    for hc in range(H // FCH):
        h0 = hc * FCH
        for c in range(Cf):
            fs_ref[c * FS:c * FS + FCH, :] = \
                fm_ref[0, c * H + h0:c * H + h0 + FCH, :]
        for hl in range(FCH):
            y = h0 + hl
            put(C, Cf, (y + 1) * W, fs_ref[hl:hl + Cf * FS:FS, :], W)


def _fused_kernel(ll_ref, lh_ref, hl_ref, hh_ref, fm_ref, wp_ref, wm_ref,
                  b_ref, t_ref, m_ref, o_ref, eo_ref, fs_ref, xf_ref, *,
                  C, Cf, Hh, Wh, R, CHK):
    W = 2 * Wh
    Cin = C + Cf

    @pl.when(pl.program_id(1) == 0)
    def _build():
        _build_input(ll_ref, lh_ref, hl_ref, hh_ref, fm_ref, t_ref,
                     eo_ref, fs_ref, xf_ref, C=C, Cf=Cf, Hh=Hh, Wh=Wh,
                     CHK=CHK)

    bias = b_ref[...]
    mask_l = m_ref[0:1, :]
    mask_r = m_ref[1:2, :]

    hb = pl.program_id(1)
    off = pl.multiple_of(hb * (R * W), W)
    opnd_pair = xf_ref[:, pl.ds(off, R * W)]
    opnd_mid = xf_ref[0:Cin, pl.ds(off + W, R * W)]
    acc = None
    for kw in range(3):
        t = (jnp.dot(wp_ref[kw], opnd_pair,
                     preferred_element_type=jnp.float32)
             + jnp.dot(wm_ref[kw], opnd_mid,
                       preferred_element_type=jnp.float32))
        if kw == 0:
            t = jnp.roll(t, 1, axis=1) * mask_l
        elif kw == 2:
            t = jnp.roll(t, -1, axis=1) * mask_r
        acc = t if acc is None else acc + t
    o_ref[0, :, :] = jnp.maximum(acc + bias, 0.0)


def kernel(LL, LH, HL, HH, fm, conv_w, conv_b):
    N, C, Hh, Wh = LL.shape
    H, W = 2 * Hh, 2 * Wh
    Nf, Cf, Hf, Wf = fm.shape
    assert (Nf, Hf, Wf) == (N, H, W)
    Cout, Cin, kh, kw = conv_w.shape
    assert (kh, kw) == (3, 3) and Cin == C + Cf

    ll2 = LL.reshape(N, C * Hh, Wh).astype(jnp.float32)
    lh2 = LH.reshape(N, C * Hh, Wh).astype(jnp.float32)
    hl2 = HL.reshape(N, C * Hh, Wh).astype(jnp.float32)
    hh2 = HH.reshape(N, C * Hh, Wh).astype(jnp.float32)
    fm2 = fm.reshape(N, Cf * H, W).astype(jnp.float32)

    wt = jnp.asarray(conv_w, jnp.float32)
    wpair = jnp.concatenate([wt[:, :, 0, :], wt[:, :, 2, :]], axis=1)
    wpair = wpair.transpose(2, 0, 1)
    wmid = wt[:, :, 1, :].transpose(2, 0, 1)
    b2 = jnp.asarray(conv_b, jnp.float32).reshape(Cout, 1)

    coef = np.array([[.5, -.5, -.5, .5],
                     [.5, -.5, .5, -.5],
                     [.5, .5, -.5, -.5],
                     [.5, .5, .5, .5]], np.float32)
    T = np.zeros((4 * Wh, 2 * W), np.float32)
    j = np.arange(Wh)
    for s in range(4):
        T[s * Wh + j, 2 * j] = coef[0, s]
        T[s * Wh + j, 2 * j + 1] = coef[1, s]
        T[s * Wh + j, W + 2 * j] = coef[2, s]
        T[s * Wh + j, W + 2 * j + 1] = coef[3, s]
    T = jnp.asarray(T)

    R = 16 if H % 16 == 0 else (8 if H % 8 == 0 else H)
    CHK = max(1, min(C, 256 // Hh))
    while C % CHK:
        CHK -= 1

    lane = np.arange(R * W) % W
    masks = np.stack([(lane != 0).astype(np.float32),
                      (lane != W - 1).astype(np.float32)])
    masks = jnp.asarray(masks)

    kernel_fn = functools.partial(_fused_kernel, C=C, Cf=Cf, Hh=Hh, Wh=Wh,
                                  R=R, CHK=CHK)
    sub_spec = pl.BlockSpec((1, C * Hh, Wh), lambda n, h: (n, 0, 0))
    out = pl.pallas_call(
        kernel_fn,
        out_shape=jax.ShapeDtypeStruct((N, Cout, H * W), jnp.float32),
        grid=(N, H // R),
        in_specs=[sub_spec, sub_spec, sub_spec, sub_spec,
                  pl.BlockSpec((1, Cf * H, W), lambda n, h: (n, 0, 0)),
                  pl.BlockSpec((3, Cout, 2 * Cin), lambda n, h: (0, 0, 0)),
                  pl.BlockSpec((3, Cout, Cin), lambda n, h: (0, 0, 0)),
                  pl.BlockSpec((Cout, 1), lambda n, h: (0, 0)),
                  pl.BlockSpec((4 * Wh, 2 * W), lambda n, h: (0, 0)),
                  pl.BlockSpec((2, R * W), lambda n, h: (0, 0))],
        out_specs=pl.BlockSpec((1, Cout, R * W), lambda n, h: (n, 0, h)),
        scratch_shapes=[pltpu.VMEM((CHK * (2 * Hh + 2), W), jnp.float32),
                        pltpu.VMEM((Cf * 17, W), jnp.float32),
                        pltpu.VMEM((2 * (C + Cf), (H + 2) * W), jnp.float32)],
        compiler_params=pltpu.CompilerParams(
            dimension_semantics=("parallel", "arbitrary")),
    )(ll2, lh2, hl2, hh2, fm2, wpair, wmid, b2, T, masks)
    return out.reshape(N, Cout, H, W)


# speedup vs baseline: 1.2259x; 1.0166x over previous
import functools

import jax
import jax.numpy as jnp
import numpy as np
from jax.experimental import pallas as pl
from jax.experimental.pallas import tpu as pltpu


def _build_input(ll_ref, lh_ref, hl_ref, hh_ref, fm_ref, t_ref,
                 eo_ref, fs_ref, xf_ref, *, C, Cf, Hh, Wh, CHK):
    W = 2 * Wh
    H = 2 * Hh
    Cin = C + Cf
    SCH = 2 * Hh + 2

    zrow = jnp.zeros((Cin, W), jnp.float32)
    xf_ref[0:Cin, 0:W] = zrow
    xf_ref[0:Cin, (H + 1) * W:] = zrow

    sb_refs = (ll_ref, lh_ref, hl_ref, hh_ref)
    for ch in range(C // CHK):
        r0 = ch * CHK * Hh
        acc = None
        for s in range(4):
            x = sb_refs[s][0, r0:r0 + CHK * Hh, :]
            t = jnp.dot(x, t_ref[s * Wh:(s + 1) * Wh, :],
                        preferred_element_type=jnp.float32)
            acc = t if acc is None else acc + t
        for cl in range(CHK):
            rows = slice(cl * Hh, (cl + 1) * Hh)
            eo_ref[cl * SCH:cl * SCH + Hh, :] = acc[rows, 0:W]
            eo_ref[cl * SCH + Hh:cl * SCH + 2 * Hh, :] = acc[rows, W:2 * W]
        c0 = ch * CHK
        for i in range(Hh):
            off = (2 * i + 1) * W
            xf_ref[c0:c0 + CHK, off:off + W] = \
                eo_ref[i:i + CHK * SCH:SCH, :]
            xf_ref[c0:c0 + CHK, off + W:off + 2 * W] = \
                eo_ref[Hh + i:Hh + i + CHK * SCH:SCH, :]

    FCH = 16
    FS = FCH + 1
    for hc in range(H // FCH):
        h0 = hc * FCH
        for c in range(Cf):
            fs_ref[c * FS:c * FS + FCH, :] = \
                fm_ref[0, c * H + h0:c * H + h0 + FCH, :]
        for hl in range(FCH):
            y = h0 + hl
            xf_ref[C:Cin, (y + 1) * W:(y + 2) * W] = \
                fs_ref[hl:hl + Cf * FS:FS, :]

    Q = (H * W) // 4
    for q in range(4):
        xf_ref[Cin:2 * Cin, q * Q:(q + 1) * Q] = \
            xf_ref[0:Cin, 2 * W + q * Q:2 * W + (q + 1) * Q]


def _fused_kernel(ll_ref, lh_ref, hl_ref, hh_ref, fm_ref, wp_ref, wm_ref,
                  b_ref, t_ref, m_ref, o_ref, eo_ref, fs_ref, xf_ref, *,
                  C, Cf, Hh, Wh, R, CHK):
    W = 2 * Wh
    Cin = C + Cf

    @pl.when(pl.program_id(1) == 0)
    def _build():
        _build_input(ll_ref, lh_ref, hl_ref, hh_ref, fm_ref, t_ref,
                     eo_ref, fs_ref, xf_ref, C=C, Cf=Cf, Hh=Hh, Wh=Wh,
                     CHK=CHK)

    bias = b_ref[...]
    mask_l = m_ref[0:1, :]
    mask_r = m_ref[1:2, :]

    hb = pl.program_id(1)
    off = pl.multiple_of(hb * (R * W), W)
    opnd_pair = xf_ref[:, pl.ds(off, R * W)]
    opnd_mid = xf_ref[0:Cin, pl.ds(off + W, R * W)]
    acc = None
    for kw in range(3):
        t = (jnp.dot(wp_ref[kw], opnd_pair,
                     preferred_element_type=jnp.float32)
             + jnp.dot(wm_ref[kw], opnd_mid,
                       preferred_element_type=jnp.float32))
        if kw == 0:
            t = jnp.roll(t, 1, axis=1) * mask_l
        elif kw == 2:
            t = jnp.roll(t, -1, axis=1) * mask_r
        acc = t if acc is None else acc + t
    o_ref[0, :, :] = jnp.maximum(acc + bias, 0.0)


def kernel(LL, LH, HL, HH, fm, conv_w, conv_b):
    N, C, Hh, Wh = LL.shape
    H, W = 2 * Hh, 2 * Wh
    Nf, Cf, Hf, Wf = fm.shape
    assert (Nf, Hf, Wf) == (N, H, W)
    Cout, Cin, kh, kw = conv_w.shape
    assert (kh, kw) == (3, 3) and Cin == C + Cf

    ll2 = LL.reshape(N, C * Hh, Wh).astype(jnp.float32)
    lh2 = LH.reshape(N, C * Hh, Wh).astype(jnp.float32)
    hl2 = HL.reshape(N, C * Hh, Wh).astype(jnp.float32)
    hh2 = HH.reshape(N, C * Hh, Wh).astype(jnp.float32)
    fm2 = fm.reshape(N, Cf * H, W).astype(jnp.float32)

    wt = jnp.asarray(conv_w, jnp.float32)
    wpair = jnp.concatenate([wt[:, :, 0, :], wt[:, :, 2, :]], axis=1)
    wpair = wpair.transpose(2, 0, 1)
    wmid = wt[:, :, 1, :].transpose(2, 0, 1)
    b2 = jnp.asarray(conv_b, jnp.float32).reshape(Cout, 1)

    coef = np.array([[.5, -.5, -.5, .5],
                     [.5, -.5, .5, -.5],
                     [.5, .5, -.5, -.5],
                     [.5, .5, .5, .5]], np.float32)
    T = np.zeros((4 * Wh, 2 * W), np.float32)
    j = np.arange(Wh)
    for s in range(4):
        T[s * Wh + j, 2 * j] = coef[0, s]
        T[s * Wh + j, 2 * j + 1] = coef[1, s]
        T[s * Wh + j, W + 2 * j] = coef[2, s]
        T[s * Wh + j, W + 2 * j + 1] = coef[3, s]
    T = jnp.asarray(T)

    R = 16 if H % 16 == 0 else (8 if H % 8 == 0 else H)
    CHK = max(1, min(C, 512 // Hh))
    while C % CHK:
        CHK -= 1

    lane = np.arange(R * W) % W
    masks = np.stack([(lane != 0).astype(np.float32),
                      (lane != W - 1).astype(np.float32)])
    masks = jnp.asarray(masks)

    kernel_fn = functools.partial(_fused_kernel, C=C, Cf=Cf, Hh=Hh, Wh=Wh,
                                  R=R, CHK=CHK)
    sub_spec = pl.BlockSpec((1, C * Hh, Wh), lambda n, h: (n, 0, 0))
    out = pl.pallas_call(
        kernel_fn,
        out_shape=jax.ShapeDtypeStruct((N, Cout, H * W), jnp.float32),
        grid=(N, H // R),
        in_specs=[sub_spec, sub_spec, sub_spec, sub_spec,
                  pl.BlockSpec((1, Cf * H, W), lambda n, h: (n, 0, 0)),
                  pl.BlockSpec((3, Cout, 2 * Cin), lambda n, h: (0, 0, 0)),
                  pl.BlockSpec((3, Cout, Cin), lambda n, h: (0, 0, 0)),
                  pl.BlockSpec((Cout, 1), lambda n, h: (0, 0)),
                  pl.BlockSpec((4 * Wh, 2 * W), lambda n, h: (0, 0)),
                  pl.BlockSpec((2, R * W), lambda n, h: (0, 0))],
        out_specs=pl.BlockSpec((1, Cout, R * W), lambda n, h: (n, 0, h)),
        scratch_shapes=[pltpu.VMEM((CHK * (2 * Hh + 2), W), jnp.float32),
                        pltpu.VMEM((Cf * 17, W), jnp.float32),
                        pltpu.VMEM((2 * (C + Cf), (H + 2) * W), jnp.float32)],
        compiler_params=pltpu.CompilerParams(
            dimension_semantics=("parallel", "arbitrary")),
    )(ll2, lh2, hl2, hh2, fm2, wpair, wmid, b2, T, masks)
    return out.reshape(N, Cout, H, W)


# speedup vs baseline: 1.2905x; 1.0527x over previous
import functools

import jax
import jax.numpy as jnp
import numpy as np
from jax.experimental import pallas as pl
from jax.experimental.pallas import tpu as pltpu


def _build_input(ll_ref, lh_ref, hl_ref, hh_ref, fm_ref, t_ref,
                 eo_ref, fs_ref, xf_ref, *, C, Cf, Hh, Wh, CHK):
    W = 2 * Wh
    H = 2 * Hh
    Cin = C + Cf
    SCH = 2 * Hh + 2

    zrow = jnp.zeros((Cin, W), jnp.float32)
    xf_ref[0:Cin, 0:W] = zrow
    xf_ref[0:Cin, (H + 1) * W:] = zrow

    sb_refs = (ll_ref, lh_ref, hl_ref, hh_ref)
    for ch in range(C // CHK):
        r0 = ch * CHK * Hh
        acc = None
        for s in range(4):
            x = sb_refs[s][0, r0:r0 + CHK * Hh, :]
            t = jnp.dot(x, t_ref[s * Wh:(s + 1) * Wh, :],
                        preferred_element_type=jnp.float32)
            acc = t if acc is None else acc + t
        for cl in range(CHK):
            rows = slice(cl * Hh, (cl + 1) * Hh)
            eo_ref[cl * SCH:cl * SCH + Hh, :] = acc[rows, 0:W]
            eo_ref[cl * SCH + Hh:cl * SCH + 2 * Hh, :] = acc[rows, W:2 * W]
        c0 = ch * CHK
        for i in range(Hh):
            off = (2 * i + 1) * W
            xf_ref[c0:c0 + CHK, off:off + W] = \
                eo_ref[i:i + CHK * SCH:SCH, :]
            xf_ref[c0:c0 + CHK, off + W:off + 2 * W] = \
                eo_ref[Hh + i:Hh + i + CHK * SCH:SCH, :]

    FCH = 16
    FS = FCH + 1
    for hc in range(H // FCH):
        h0 = hc * FCH
        for c in range(Cf):
            fs_ref[c * FS:c * FS + FCH, :] = \
                fm_ref[0, c * H + h0:c * H + h0 + FCH, :]
        for hl in range(FCH):
            y = h0 + hl
            xf_ref[C:Cin, (y + 1) * W:(y + 2) * W] = \
                fs_ref[hl:hl + Cf * FS:FS, :]

    Q = (H * W) // 4
    for q in range(4):
        xf_ref[Cin:2 * Cin, q * Q:(q + 1) * Q] = \
            xf_ref[0:Cin, 2 * W + q * Q:2 * W + (q + 1) * Q]


def _fused_kernel(ll_ref, lh_ref, hl_ref, hh_ref, fm_ref, wp_ref, wm_ref,
                  b_ref, t_ref, m_ref, o_ref, eo_ref, fs_ref, xf_ref, *,
                  C, Cf, Hh, Wh, R, CHK):
    W = 2 * Wh
    Cin = C + Cf

    @pl.when(pl.program_id(1) == 0)
    def _build():
        _build_input(ll_ref, lh_ref, hl_ref, hh_ref, fm_ref, t_ref,
                     eo_ref, fs_ref, xf_ref, C=C, Cf=Cf, Hh=Hh, Wh=Wh,
                     CHK=CHK)

    bias = b_ref[...]
    mask_l = m_ref[0:1, :]
    mask_r = m_ref[1:2, :]

    hb = pl.program_id(1)
    off = pl.multiple_of(hb * (R * W), W)
    opnd_pair = xf_ref[:, pl.ds(off, R * W)]
    opnd_mid = xf_ref[0:Cin, pl.ds(off + W, R * W)]
    acc = None
    for kw in range(3):
        t = (jnp.dot(wp_ref[kw], opnd_pair,
                     preferred_element_type=jnp.float32)
             + jnp.dot(wm_ref[kw], opnd_mid,
                       preferred_element_type=jnp.float32))
        if kw == 0:
            t = jnp.roll(t, 1, axis=1) * mask_l
        elif kw == 2:
            t = jnp.roll(t, -1, axis=1) * mask_r
        acc = t if acc is None else acc + t
    o_ref[0, :, :] = jnp.maximum(acc + bias, 0.0)


def kernel(LL, LH, HL, HH, fm, conv_w, conv_b):
    N, C, Hh, Wh = LL.shape
    H, W = 2 * Hh, 2 * Wh
    Nf, Cf, Hf, Wf = fm.shape
    assert (Nf, Hf, Wf) == (N, H, W)
    Cout, Cin, kh, kw = conv_w.shape
    assert (kh, kw) == (3, 3) and Cin == C + Cf

    ll2 = LL.reshape(N, C * Hh, Wh).astype(jnp.float32)
    lh2 = LH.reshape(N, C * Hh, Wh).astype(jnp.float32)
    hl2 = HL.reshape(N, C * Hh, Wh).astype(jnp.float32)
    hh2 = HH.reshape(N, C * Hh, Wh).astype(jnp.float32)
    fm2 = fm.reshape(N, Cf * H, W).astype(jnp.float32)

    wt = jnp.asarray(conv_w, jnp.float32)
    wpair = jnp.concatenate([wt[:, :, 0, :], wt[:, :, 2, :]], axis=1)
    wpair = wpair.transpose(2, 0, 1)
    wmid = wt[:, :, 1, :].transpose(2, 0, 1)
    b2 = jnp.asarray(conv_b, jnp.float32).reshape(Cout, 1)

    coef = np.array([[.5, -.5, -.5, .5],
                     [.5, -.5, .5, -.5],
                     [.5, .5, -.5, -.5],
                     [.5, .5, .5, .5]], np.float32)
    T = np.zeros((4 * Wh, 2 * W), np.float32)
    j = np.arange(Wh)
    for s in range(4):
        T[s * Wh + j, 2 * j] = coef[0, s]
        T[s * Wh + j, 2 * j + 1] = coef[1, s]
        T[s * Wh + j, W + 2 * j] = coef[2, s]
        T[s * Wh + j, W + 2 * j + 1] = coef[3, s]
    T = jnp.asarray(T)

    R = 32 if H % 32 == 0 else (16 if H % 16 == 0 else H)
    CHK = max(1, min(C, 512 // Hh))
    while C % CHK:
        CHK -= 1

    lane = np.arange(R * W) % W
    masks = np.stack([(lane != 0).astype(np.float32),
                      (lane != W - 1).astype(np.float32)])
    masks = jnp.asarray(masks)

    kernel_fn = functools.partial(_fused_kernel, C=C, Cf=Cf, Hh=Hh, Wh=Wh,
                                  R=R, CHK=CHK)
    sub_spec = pl.BlockSpec((1, C * Hh, Wh), lambda n, h: (n, 0, 0))
    out = pl.pallas_call(
        kernel_fn,
        out_shape=jax.ShapeDtypeStruct((N, Cout, H * W), jnp.float32),
        grid=(N, H // R),
        in_specs=[sub_spec, sub_spec, sub_spec, sub_spec,
                  pl.BlockSpec((1, Cf * H, W), lambda n, h: (n, 0, 0)),
                  pl.BlockSpec((3, Cout, 2 * Cin), lambda n, h: (0, 0, 0)),
                  pl.BlockSpec((3, Cout, Cin), lambda n, h: (0, 0, 0)),
                  pl.BlockSpec((Cout, 1), lambda n, h: (0, 0)),
                  pl.BlockSpec((4 * Wh, 2 * W), lambda n, h: (0, 0)),
                  pl.BlockSpec((2, R * W), lambda n, h: (0, 0))],
        out_specs=pl.BlockSpec((1, Cout, R * W), lambda n, h: (n, 0, h)),
        scratch_shapes=[pltpu.VMEM((CHK * (2 * Hh + 2), W), jnp.float32),
                        pltpu.VMEM((Cf * 17, W), jnp.float32),
                        pltpu.VMEM((2 * (C + Cf), (H + 2) * W), jnp.float32)],
        compiler_params=pltpu.CompilerParams(
            dimension_semantics=("parallel", "arbitrary")),
    )(ll2, lh2, hl2, hh2, fm2, wpair, wmid, b2, T, masks)
    return out.reshape(N, Cout, H, W)
